# baseline (speedup 1.0000x reference)
import functools

import jax
import jax.numpy as jnp
from jax import lax
from jax.experimental import pallas as pl
from jax.experimental.pallas import tpu as pltpu
from jax.experimental.pallas import tpu_sc as plsc

_NW = 32
_CHUNK = 128


def _sc_gather(table, ids_flat, out_wd=None):
    ntok = ids_flat.shape[0]
    wd = table.shape[1]
    out_wd = wd if out_wd is None else out_wd
    dt = table.dtype
    tok_per_w = ntok // _NW
    n_chunk = tok_per_w // _CHUNK
    mesh = plsc.VectorSubcoreMesh(core_axis_name="c", subcore_axis_name="s")

    nb = 4

    @functools.partial(
        pl.kernel,
        mesh=mesh,
        out_type=jax.ShapeDtypeStruct((ntok, out_wd), dt),
        scratch_types=[
            pltpu.VMEM((n_chunk, _CHUNK), jnp.int32),
            pltpu.VMEM((nb, _CHUNK, wd), dt),
            pltpu.SemaphoreType.DMA,
            pltpu.SemaphoreType.DMA,
        ],
    )
    def k(table_hbm, idx_hbm, out_hbm, idx_v, rows_v, gsem, osem):
        wid = lax.axis_index("s") * 2 + lax.axis_index("c")
        base = wid * tok_per_w

        pltpu.sync_copy(idx_hbm.at[wid], idx_v)

        def gath(g, slot):
            pltpu.async_copy(table_hbm.at[idx_v.at[g]], rows_v.at[slot], gsem)

        def gath_wait(g, slot):
            pltpu.make_async_copy(table_hbm.at[idx_v.at[g]],
                                  rows_v.at[slot], gsem).wait()

        owd = out_hbm.shape[1]

        def wr(g, slot):
            pltpu.async_copy(rows_v.at[slot, :, pl.ds(0, owd)],
                             out_hbm.at[pl.ds(base + g * _CHUNK, _CHUNK)], osem)

        def wr_wait(g, slot):
            pltpu.make_async_copy(
                rows_v.at[slot, :, pl.ds(0, owd)],
                out_hbm.at[pl.ds(base + g * _CHUNK, _CHUNK)], osem).wait()

        for p in range(nb - 1):
            gath(p, p)

        def body(gg, _):
            for b in range(nb):
                g = gg * nb + b
                gath_wait(g, b)
                wr(g, b)
                @pl.when(g > 0)
                def _():
                    wr_wait(g - 1, (b + nb - 1) % nb)

                @pl.when(g + nb - 1 < n_chunk)
                def _():
                    gath(g + nb - 1, (b + nb - 1) % nb)
            return 0

        lax.fori_loop(0, n_chunk // nb, body, 0)
        wr_wait(n_chunk - 1, nb - 1)

    return k(table, ids_flat.reshape(_NW, n_chunk, _CHUNK))


_BT = 8192


def _dense_body(g_ref, s_ref, w_ref, pt_ref, gm_ref, bt_ref, o_ref):
    _, dim, bt = o_ref.shape
    g = g_ref[...]
    xt = lax.dot_general(w_ref[...], g, (((0,), (1,)), ((), ())),
                         preferred_element_type=jnp.float32)
    l = pl.program_id(0)
    s = s_ref[...].reshape(1, bt)
    ptid = lax.broadcast_in_dim(l * 3 + s, (64, bt), (0, 1))
    cls = lax.broadcasted_iota(jnp.int32, (64, bt), 0)
    oht = jnp.where(ptid == cls, 1.0, 0.0)
    xt = xt + lax.dot_general(pt_ref[...], oht, (((0,), (0,)), ((), ())),
                              preferred_element_type=jnp.float32)
    mean = jnp.sum(xt, axis=0, keepdims=True) * (1.0 / dim)
    xc = xt - mean
    var = jnp.sum(xc * xc, axis=0, keepdims=True) * (1.0 / dim)
    y = xc * lax.rsqrt(var + 1e-12)
    y = y * gm_ref[...] + bt_ref[...]
    o_ref[...] = y.reshape(1, dim, bt)


def _dense_body_acc(prev_ref, g_ref, s_ref, w_ref, pt_ref, gm_ref, bt_ref, o_ref):
    del prev_ref
    _dense_body(g_ref, s_ref, w_ref, pt_ref, gm_ref, bt_ref, o_ref)


def _tc_dense(g2, seg3, W2, PTa, gammaT, betaT, batch, L, b_off=0, prev=None):
    wd = W2.shape[0]
    dim = W2.shape[1]
    hb = seg3.shape[2]
    nbb = hb // _BT
    grid = (L, nbb)
    in_specs = [
        pl.BlockSpec((_BT, wd), lambda l, j: (l * nbb + j, 0)),
        pl.BlockSpec((1, 1, _BT), lambda l, j: (l, 0, j)),
        pl.BlockSpec((wd, dim), lambda l, j: (0, 0)),
        pl.BlockSpec((64, dim), lambda l, j: (0, 0)),
        pl.BlockSpec((dim, 1), lambda l, j: (0, 0)),
        pl.BlockSpec((dim, 1), lambda l, j: (0, 0)),
    ]
    args = (g2, seg3, W2, PTa, gammaT, betaT)
    body = _dense_body
    kwargs = {}
    if prev is not None:
        in_specs = [pl.BlockSpec(memory_space=pl.ANY)] + in_specs
        args = (prev,) + args
        body = _dense_body_acc
        kwargs["input_output_aliases"] = {0: 0}
    return pl.pallas_call(
        body,
        grid=grid,
        in_specs=in_specs,
        out_specs=pl.BlockSpec((1, dim, _BT), lambda l, j: (l, 0, j + b_off)),
        out_shape=jax.ShapeDtypeStruct((L, dim, batch), jnp.float32),
        **kwargs,
    )(*args)


def kernel(input_ids, segment_ids, word_emb, W2, pos_emb, type_emb, gamma, beta):
    batch, L = input_ids.shape
    dim = W2.shape[1]
    idsT = input_ids.astype(jnp.int32).T
    segT = segment_ids.astype(jnp.int32).T
    gammaT = gamma.reshape(-1, 1)
    betaT = beta.reshape(-1, 1)

    PTa = jnp.zeros((64, dim), jnp.float32)
    PTa = PTa.at[: 3 * L].set(
        (pos_emb[:, None, :] + type_emb[None, :, :]).reshape(3 * L, dim))

    nsplit = 2
    hb = batch // nsplit
    out = None
    for q in range(nsplit):
        ids_q = idsT[:, q * hb:(q + 1) * hb].reshape(-1)
        g_q = _sc_gather(word_emb, ids_q)
        seg3_q = segT[:, q * hb:(q + 1) * hb].reshape(L, 1, hb)
        out = _tc_dense(g_q, seg3_q, W2, PTa, gammaT, betaT, batch, L,
                        b_off=q * (hb // _BT), prev=out)
    return jnp.transpose(out, (2, 0, 1))


# speedup vs baseline: 1.0108x; 1.0108x over previous
import functools

import jax
import jax.numpy as jnp
from jax import lax
from jax.experimental import pallas as pl
from jax.experimental.pallas import tpu as pltpu
from jax.experimental.pallas import tpu_sc as plsc

_NW = 32
_CHUNK = 128


def _sc_gather(table, ids_flat, out_wd=None):
    ntok = ids_flat.shape[0]
    wd = table.shape[1]
    out_wd = wd if out_wd is None else out_wd
    dt = table.dtype
    tok_per_w = ntok // _NW
    n_chunk = tok_per_w // _CHUNK
    mesh = plsc.VectorSubcoreMesh(core_axis_name="c", subcore_axis_name="s")

    nb = 4

    @functools.partial(
        pl.kernel,
        mesh=mesh,
        out_type=jax.ShapeDtypeStruct((ntok, out_wd), dt),
        scratch_types=[
            pltpu.VMEM((n_chunk, _CHUNK), jnp.int32),
            pltpu.VMEM((nb, _CHUNK, wd), dt),
            pltpu.SemaphoreType.DMA,
            pltpu.SemaphoreType.DMA,
        ],
    )
    def k(table_hbm, idx_hbm, out_hbm, idx_v, rows_v, gsem, osem):
        wid = lax.axis_index("s") * 2 + lax.axis_index("c")
        base = wid * tok_per_w

        pltpu.sync_copy(idx_hbm.at[wid], idx_v)

        def gath(g, slot):
            pltpu.async_copy(table_hbm.at[idx_v.at[g]], rows_v.at[slot], gsem)

        def gath_wait(g, slot):
            pltpu.make_async_copy(table_hbm.at[idx_v.at[g]],
                                  rows_v.at[slot], gsem).wait()

        owd = out_hbm.shape[1]

        def wr(g, slot):
            pltpu.async_copy(rows_v.at[slot, :, pl.ds(0, owd)],
                             out_hbm.at[pl.ds(base + g * _CHUNK, _CHUNK)], osem)

        def wr_wait(g, slot):
            pltpu.make_async_copy(
                rows_v.at[slot, :, pl.ds(0, owd)],
                out_hbm.at[pl.ds(base + g * _CHUNK, _CHUNK)], osem).wait()

        for p in range(nb - 1):
            gath(p, p)

        def body(gg, _):
            for b in range(nb):
                g = gg * nb + b
                gath_wait(g, b)
                wr(g, b)
                @pl.when(g > 0)
                def _():
                    wr_wait(g - 1, (b + nb - 1) % nb)

                @pl.when(g + nb - 1 < n_chunk)
                def _():
                    gath(g + nb - 1, (b + nb - 1) % nb)
            return 0

        lax.fori_loop(0, n_chunk // nb, body, 0)
        wr_wait(n_chunk - 1, nb - 1)

    return k(table, ids_flat.reshape(_NW, n_chunk, _CHUNK))


_BT = 4096


def _dense_body(g_ref, s_ref, w_ref, pt_ref, gm_ref, bt_ref, o_ref):
    _, dim, bt = o_ref.shape
    g = g_ref[...]
    xt = lax.dot_general(w_ref[...], g, (((0,), (1,)), ((), ())),
                         preferred_element_type=jnp.float32)
    l = pl.program_id(0)
    s = s_ref[...].reshape(1, bt)
    ptid = lax.broadcast_in_dim(l * 3 + s, (64, bt), (0, 1))
    cls = lax.broadcasted_iota(jnp.int32, (64, bt), 0)
    oht = jnp.where(ptid == cls, 1.0, 0.0)
    xt = xt + lax.dot_general(pt_ref[...], oht, (((0,), (0,)), ((), ())),
                              preferred_element_type=jnp.float32)
    mean = jnp.sum(xt, axis=0, keepdims=True) * (1.0 / dim)
    xc = xt - mean
    var = jnp.sum(xc * xc, axis=0, keepdims=True) * (1.0 / dim)
    y = xc * lax.rsqrt(var + 1e-12)
    y = y * gm_ref[...] + bt_ref[...]
    o_ref[...] = y.reshape(1, dim, bt)


def _dense_body_acc(prev_ref, g_ref, s_ref, w_ref, pt_ref, gm_ref, bt_ref, o_ref):
    del prev_ref
    _dense_body(g_ref, s_ref, w_ref, pt_ref, gm_ref, bt_ref, o_ref)


def _tc_dense(g2, seg3, W2, PTa, gammaT, betaT, batch, L, b_off=0, prev=None):
    wd = W2.shape[0]
    dim = W2.shape[1]
    hb = g2.shape[0] // L
    nbb = hb // _BT
    grid = (L, nbb)
    in_specs = [
        pl.BlockSpec((_BT, wd), lambda l, j: (l * nbb + j, 0)),
        pl.BlockSpec((1, 1, _BT), lambda l, j: (l, 0, j + b_off)),
        pl.BlockSpec((wd, dim), lambda l, j: (0, 0)),
        pl.BlockSpec((64, dim), lambda l, j: (0, 0)),
        pl.BlockSpec((dim, 1), lambda l, j: (0, 0)),
        pl.BlockSpec((dim, 1), lambda l, j: (0, 0)),
    ]
    args = (g2, seg3, W2, PTa, gammaT, betaT)
    body = _dense_body
    kwargs = {}
    if prev is not None:
        in_specs = [pl.BlockSpec(memory_space=pl.ANY)] + in_specs
        args = (prev,) + args
        body = _dense_body_acc
        kwargs["input_output_aliases"] = {0: 0}
    return pl.pallas_call(
        body,
        grid=grid,
        in_specs=in_specs,
        out_specs=pl.BlockSpec((1, dim, _BT), lambda l, j: (l, 0, j + b_off)),
        out_shape=jax.ShapeDtypeStruct((L, dim, batch), jnp.float32),
        **kwargs,
    )(*args)


def kernel(input_ids, segment_ids, word_emb, W2, pos_emb, type_emb, gamma, beta):
    batch, L = input_ids.shape
    dim = W2.shape[1]
    idsT = input_ids.astype(jnp.int32).T
    segT = segment_ids.astype(jnp.int32).T
    gammaT = gamma.reshape(-1, 1)
    betaT = beta.reshape(-1, 1)

    PTa = jnp.zeros((64, dim), jnp.float32)
    PTa = PTa.at[: 3 * L].set(
        (pos_emb[:, None, :] + type_emb[None, :, :]).reshape(3 * L, dim))

    nsplit = 4
    hb = batch // nsplit
    seg3 = segT.reshape(L, 1, batch)
    out = None
    for q in range(nsplit):
        ids_q = idsT[:, q * hb:(q + 1) * hb].reshape(-1)
        g_q = _sc_gather(word_emb, ids_q)
        out = _tc_dense(g_q, seg3, W2, PTa, gammaT, betaT, batch, L,
                        b_off=q * (hb // _BT), prev=out)
    return jnp.transpose(out, (2, 0, 1))
